# Initial kernel scaffold; baseline (speedup 1.0000x reference)
import jax
import jax.numpy as jnp
from jax.experimental import pallas as pl


def kernel(cat_base_ixs, cat_ante_ixs, hvb_row, hvb_col, hvb_val, hva_row, hva_col, hva_val, hvb_top, hva_top, worddists, sqworddists, corefons, cat_table, hvec_table, W1, b1, W2, b2):
    raise NotImplementedError("write your pallas kernel here")


# speedup vs baseline: 2.1280x; 2.1280x over previous
import functools

import jax
import jax.numpy as jnp
from jax import lax
from jax.experimental import pallas as pl
from jax.experimental.pallas import tpu as pltpu
from jax.experimental.pallas import tpu_sc as plsc

B = 16384
NNZ = 65536
SEMD = 32
HID = 128
OUT = 2
NC = 2
NS = 16
CH = 128
ENT_PER_TILE = NNZ // NS
NCH = ENT_PER_TILE // CH
RPT = B // NS
CAT_CH = B // NS // CH

_sc_mesh = plsc.VectorSubcoreMesh(core_axis_name="c", subcore_axis_name="s")


@functools.partial(
    pl.kernel,
    out_type=(
        jax.ShapeDtypeStruct((NC, B, SEMD), jnp.float32),
        jax.ShapeDtypeStruct((NC, B, SEMD), jnp.float32),
    ),
    mesh=_sc_mesh,
    compiler_params=pltpu.CompilerParams(use_tc_tiling_on_sc=False),
    scratch_types=[
        pltpu.VMEM((NCH, CH), jnp.int32),
        pltpu.VMEM((NCH, CH), jnp.int32),
        pltpu.VMEM((NCH, CH), jnp.float32),
        pltpu.VMEM((CH, SEMD), jnp.float32),
        pltpu.VMEM((CAT_CH, CH), jnp.int32),
        pltpu.VMEM((CH, SEMD), jnp.float32),
        pltpu.VMEM_SHARED((B, SEMD), jnp.float32),
        pltpu.SemaphoreType.DMA,
    ],
)
def _sc_embed(hv_col, hv_row, hv_val, hv_top, cat_ixs, cat_table, hvec_table,
              hv_embed, cat_embed,
              col_v, row_v, val_v, rows_v, cidx_v, crows_v, acc, sem):
    c = lax.axis_index("c")
    s = lax.axis_index("s")

    pltpu.sync_copy(hv_top.at[c, pl.ds(s * RPT, RPT)], acc.at[pl.ds(s * RPT, RPT)])
    pltpu.sync_copy(hv_col.at[c, s], col_v)
    pltpu.sync_copy(hv_row.at[c, s], row_v)
    pltpu.sync_copy(hv_val.at[c, s], val_v)
    plsc.subcore_barrier()

    def chunk_body(j, carry):
        pltpu.async_copy(hvec_table.at[col_v.at[j]], rows_v, sem).wait()
        for g in range(CH // 16):
            vv = val_v[j, pl.ds(g * 16, 16)]
            for t in range(16):
                e = g * 16 + t
                rows_v[e, pl.ds(0, 16)] = rows_v[e, pl.ds(0, 16)] * vv[t]
                rows_v[e, pl.ds(16, 16)] = rows_v[e, pl.ds(16, 16)] * vv[t]
        pltpu.sync_copy(rows_v, acc.at[row_v.at[j]], add=True)
        return carry

    lax.fori_loop(0, NCH, chunk_body, 0)

    pltpu.sync_copy(cat_ixs.at[c, s], cidx_v)

    def cat_body(q, carry):
        pltpu.async_copy(cat_table.at[cidx_v.at[q]], crows_v, sem).wait()
        pltpu.sync_copy(crows_v, cat_embed.at[c, pl.ds(s * RPT + q * CH, CH)])
        return carry

    lax.fori_loop(0, CAT_CH, cat_body, 0)

    plsc.subcore_barrier()
    pltpu.sync_copy(acc.at[pl.ds(s * RPT, RPT)], hv_embed.at[c, pl.ds(s * RPT, RPT)])


def _mlp_body(cb, ca, hb, ha, wd, sq, co, w1a, w1b, w1c, w1d, w1s, b1, w2, b2,
              out):
    hp = lax.Precision.HIGHEST
    x = jnp.dot(cb[...], w1a[...], precision=hp, preferred_element_type=jnp.float32)
    x = x + jnp.dot(ca[...], w1b[...], precision=hp, preferred_element_type=jnp.float32)
    x = x + jnp.dot(hb[...], w1c[...], precision=hp, preferred_element_type=jnp.float32)
    x = x + jnp.dot(ha[...], w1d[...], precision=hp, preferred_element_type=jnp.float32)
    w1s_ = w1s[...]
    x = x + wd[...] * w1s_[0:1, :] + sq[...] * w1s_[1:2, :] + co[...] * w1s_[2:3, :]
    x = x + b1[...]
    h = jnp.maximum(x, 0.0)
    z = jnp.dot(h, w2[...], precision=hp, preferred_element_type=jnp.float32) + b2[...]
    m = jnp.max(z, axis=1, keepdims=True)
    lse = m + jnp.log(jnp.sum(jnp.exp(z - m), axis=1, keepdims=True))
    out[...] = z - lse


def _mlp(cb, ca, hb, ha, wd, sq, co, w1a, w1b, w1c, w1d, w1s, b1, w2, b2):
    BLK = 2048
    row_block = lambda i: (i, 0)
    rep = lambda i: (0, 0)
    return pl.pallas_call(
        _mlp_body,
        grid=(B // BLK,),
        in_specs=[
            pl.BlockSpec((BLK, SEMD), row_block),
            pl.BlockSpec((BLK, SEMD), row_block),
            pl.BlockSpec((BLK, SEMD), row_block),
            pl.BlockSpec((BLK, SEMD), row_block),
            pl.BlockSpec((BLK, 1), row_block),
            pl.BlockSpec((BLK, 1), row_block),
            pl.BlockSpec((BLK, 1), row_block),
            pl.BlockSpec((SEMD, HID), rep),
            pl.BlockSpec((SEMD, HID), rep),
            pl.BlockSpec((SEMD, HID), rep),
            pl.BlockSpec((SEMD, HID), rep),
            pl.BlockSpec((3, HID), rep),
            pl.BlockSpec((1, HID), rep),
            pl.BlockSpec((HID, OUT), rep),
            pl.BlockSpec((1, OUT), rep),
        ],
        out_specs=pl.BlockSpec((BLK, OUT), row_block),
        out_shape=jax.ShapeDtypeStruct((B, OUT), jnp.float32),
    )(cb, ca, hb, ha, wd, sq, co, w1a, w1b, w1c, w1d, w1s, b1, w2, b2)


def kernel(cat_base_ixs, cat_ante_ixs, hvb_row, hvb_col, hvb_val, hva_row,
           hva_col, hva_val, hvb_top, hva_top, worddists, sqworddists,
           corefons, cat_table, hvec_table, W1, b1, W2, b2):
    hv_col = jnp.stack([hvb_col, hva_col]).astype(jnp.int32).reshape(NC, NS, NCH, CH)
    hv_row = jnp.stack([hvb_row, hva_row]).astype(jnp.int32).reshape(NC, NS, NCH, CH)
    hv_val = jnp.stack([hvb_val, hva_val]).reshape(NC, NS, NCH, CH)
    hv_top = jnp.stack([hvb_top, hva_top])
    cat_ixs = jnp.stack([cat_base_ixs, cat_ante_ixs]).astype(jnp.int32)
    cat_ixs = cat_ixs.reshape(NC, NS, CAT_CH, CH)

    hv_embed, cat_embed = _sc_embed(hv_col, hv_row, hv_val, hv_top, cat_ixs,
                                    cat_table, hvec_table)

    return _mlp(cat_embed[0], cat_embed[1], hv_embed[0], hv_embed[1],
                worddists.reshape(B, 1), sqworddists.reshape(B, 1),
                corefons.reshape(B, 1),
                W1[0:SEMD], W1[SEMD:2 * SEMD], W1[2 * SEMD:3 * SEMD],
                W1[3 * SEMD:4 * SEMD], W1[4 * SEMD:],
                b1.reshape(1, HID), W2, b2.reshape(1, OUT))
